# baseline (speedup 1.0000x reference)
import functools

import jax
import jax.numpy as jnp
from jax import lax
from jax.experimental import pallas as pl
from jax.experimental.pallas import tpu as pltpu
from jax.experimental.pallas import tpu_sc as plsc

TOP_K = 8
Z_LOSS_COEF = 0.001
NUM_CORES = 2
NUM_SUBCORES = 16
NUM_WORKERS = NUM_CORES * NUM_SUBCORES
LANES = 16
CHUNKS = (4096, 8192, 4096)
TM = 512


def _softmax_body(x_ref, w_ref, probs_ref, zsq_ref):
    logits = lax.dot_general(
        x_ref[...], w_ref[...], (((1,), (1,)), ((), ())),
        preferred_element_type=jnp.float32)
    m = jnp.max(logits, axis=-1, keepdims=True)
    e = jnp.exp(logits - m)
    s = jnp.sum(e, axis=-1, keepdims=True)
    probs_ref[...] = e / s
    z = m + jnp.log(s)
    part = jnp.reshape(jnp.sum(z * z), (1, 1))
    @pl.when(pl.program_id(0) == 0)
    def _init():
        zsq_ref[...] = part
    @pl.when(pl.program_id(0) != 0)
    def _acc():
        zsq_ref[...] += part


def _softmax_tc_chunk(x_flat, w, tok_base, chunk_tokens):
    h = x_flat.shape[1]
    e_dim = w.shape[0]
    steps = chunk_tokens // TM
    base = tok_base // TM
    probs, zsq = pl.pallas_call(
        _softmax_body,
        grid=(steps,),
        in_specs=[
            pl.BlockSpec((TM, h), lambda i: (base + i, 0)),
            pl.BlockSpec((e_dim, h), lambda i: (0, 0)),
        ],
        out_specs=[
            pl.BlockSpec((TM, e_dim), lambda i: (i, 0)),
            pl.BlockSpec((1, 1), lambda i: (0, 0)),
        ],
        out_shape=[
            jax.ShapeDtypeStruct((chunk_tokens, e_dim), jnp.float32),
            jax.ShapeDtypeStruct((1, 1), jnp.float32),
        ],
    )(x_flat, w)
    return probs, zsq


def _merge16(ka, va, kb, vb):
    krb = lax.rev(kb, (0,))
    vrb = lax.rev(vb, (0,))
    c = ka >= krb
    tk = jnp.where(c, ka, krb)
    tv = jnp.where(c, va, vrb)
    return plsc.sort_key_val(tk, tv, descending=True)


def _make_sc_topk(t, e_dim):
    rows_per_w = t // NUM_WORKERS
    mesh = plsc.VectorSubcoreMesh(
        core_axis_name="c", subcore_axis_name="s")

    @functools.partial(
        pl.kernel,
        out_type=[
            jax.ShapeDtypeStruct((t, TOP_K), jnp.int32),
            jax.ShapeDtypeStruct((t, TOP_K), jnp.float32),
        ],
        mesh=mesh,
        compiler_params=pltpu.CompilerParams(needs_layout_passes=False),
        scratch_types=[
            pltpu.VMEM((rows_per_w, e_dim), jnp.float32),
            pltpu.VMEM((rows_per_w, TOP_K), jnp.int32),
            pltpu.VMEM((rows_per_w, TOP_K), jnp.float32),
        ],
    )
    def sc_topk(probs_hbm, idx_hbm, val_hbm, pbuf, ibuf, vbuf):
        wid = lax.axis_index("s") * NUM_CORES + lax.axis_index("c")
        pltpu.sync_copy(
            probs_hbm.at[pl.ds(wid * rows_per_w, rows_per_w), :],
            pbuf)
        iot = lax.iota(jnp.int32, LANES)
        msk = iot < TOP_K

        @plsc.parallel_loop(0, rows_per_w, 1, unroll=4)
        def row(r):
            ks, vs = [], []
            for j in range(e_dim // LANES):
                kj, vj = plsc.sort_key_val(
                    pbuf[r, pl.ds(j * LANES, LANES)], iot + j * LANES,
                    descending=True)
                ks.append(kj)
                vs.append(vj)
            k01, v01 = _merge16(ks[0], vs[0], ks[1], vs[1])
            k23, v23 = _merge16(ks[2], vs[2], ks[3], vs[3])
            kt, vt = _merge16(k01, v01, k23, v23)
            ssum = jnp.sum(jnp.where(msk, kt, 0.0))
            vn = kt / (ssum + 1e-9)
            rvec = jnp.full((LANES,), r, jnp.int32)
            plsc.store_scatter(vbuf, [rvec, iot], vn, mask=msk)
            plsc.store_scatter(ibuf, [rvec, iot], vt, mask=msk)

        out_base = wid * rows_per_w
        pltpu.sync_copy(ibuf, idx_hbm.at[pl.ds(out_base, rows_per_w), :])
        pltpu.sync_copy(vbuf, val_hbm.at[pl.ds(out_base, rows_per_w), :])

    return sc_topk


@jax.jit
def _router(x_flat, w):
    t, _ = x_flat.shape
    e_dim = w.shape[0]
    sc_topk = {n: _make_sc_topk(n, e_dim) for n in set(CHUNKS)}
    probs_c, idx_c, val_c, zsq_c = [], [], [], []
    tok_base = 0
    for chunk_tokens in CHUNKS:
        p, zs = _softmax_tc_chunk(x_flat, w, tok_base, chunk_tokens)
        tok_base += chunk_tokens
        i_f, v_f = sc_topk[chunk_tokens](p)
        probs_c.append(p)
        idx_c.append(i_f)
        val_c.append(v_f)
        zsq_c.append(zs)
    probs = jnp.concatenate(probs_c, axis=0)
    idx = jnp.concatenate(idx_c, axis=0)
    val = jnp.concatenate(val_c, axis=0)
    zsq = sum(z[0, 0] for z in zsq_c)
    z_loss = Z_LOSS_COEF * zsq / t
    return idx, val, probs, z_loss


def kernel(x_flat, W):
    return _router(x_flat, W)


# speedup vs baseline: 1.0089x; 1.0089x over previous
import functools

import jax
import jax.numpy as jnp
from jax import lax
from jax.experimental import pallas as pl
from jax.experimental.pallas import tpu as pltpu
from jax.experimental.pallas import tpu_sc as plsc

TOP_K = 8
Z_LOSS_COEF = 0.001
NUM_CORES = 2
NUM_SUBCORES = 16
NUM_WORKERS = NUM_CORES * NUM_SUBCORES
LANES = 16
CHUNKS = (6144, 6144, 4096)
TM = 512


def _softmax_body(x_ref, w_ref, probs_ref, zsq_ref):
    logits = lax.dot_general(
        x_ref[...], w_ref[...], (((1,), (1,)), ((), ())),
        preferred_element_type=jnp.float32)
    m = jnp.max(logits, axis=-1, keepdims=True)
    e = jnp.exp(logits - m)
    s = jnp.sum(e, axis=-1, keepdims=True)
    probs_ref[...] = e / s
    z = m + jnp.log(s)
    part = jnp.reshape(jnp.sum(z * z), (1, 1))
    @pl.when(pl.program_id(0) == 0)
    def _init():
        zsq_ref[...] = part
    @pl.when(pl.program_id(0) != 0)
    def _acc():
        zsq_ref[...] += part


def _softmax_tc_chunk(x_flat, w, tok_base, chunk_tokens):
    h = x_flat.shape[1]
    e_dim = w.shape[0]
    steps = chunk_tokens // TM
    base = tok_base // TM
    probs, zsq = pl.pallas_call(
        _softmax_body,
        grid=(steps,),
        in_specs=[
            pl.BlockSpec((TM, h), lambda i: (base + i, 0)),
            pl.BlockSpec((e_dim, h), lambda i: (0, 0)),
        ],
        out_specs=[
            pl.BlockSpec((TM, e_dim), lambda i: (i, 0)),
            pl.BlockSpec((1, 1), lambda i: (0, 0)),
        ],
        out_shape=[
            jax.ShapeDtypeStruct((chunk_tokens, e_dim), jnp.float32),
            jax.ShapeDtypeStruct((1, 1), jnp.float32),
        ],
    )(x_flat, w)
    return probs, zsq


def _merge16(ka, va, kb, vb):
    krb = lax.rev(kb, (0,))
    vrb = lax.rev(vb, (0,))
    c = ka >= krb
    tk = jnp.where(c, ka, krb)
    tv = jnp.where(c, va, vrb)
    return plsc.sort_key_val(tk, tv, descending=True)


def _make_sc_topk(t, e_dim):
    rows_per_w = t // NUM_WORKERS
    mesh = plsc.VectorSubcoreMesh(
        core_axis_name="c", subcore_axis_name="s")

    @functools.partial(
        pl.kernel,
        out_type=[
            jax.ShapeDtypeStruct((t, TOP_K), jnp.int32),
            jax.ShapeDtypeStruct((t, TOP_K), jnp.float32),
        ],
        mesh=mesh,
        compiler_params=pltpu.CompilerParams(needs_layout_passes=False),
        scratch_types=[
            pltpu.VMEM((rows_per_w, e_dim), jnp.float32),
            pltpu.VMEM((rows_per_w, TOP_K), jnp.int32),
            pltpu.VMEM((rows_per_w, TOP_K), jnp.float32),
        ],
    )
    def sc_topk(probs_hbm, idx_hbm, val_hbm, pbuf, ibuf, vbuf):
        wid = lax.axis_index("s") * NUM_CORES + lax.axis_index("c")
        pltpu.sync_copy(
            probs_hbm.at[pl.ds(wid * rows_per_w, rows_per_w), :],
            pbuf)
        iot = lax.iota(jnp.int32, LANES)
        msk = iot < TOP_K

        @plsc.parallel_loop(0, rows_per_w, 1, unroll=4)
        def row(r):
            ks, vs = [], []
            for j in range(e_dim // LANES):
                kj, vj = plsc.sort_key_val(
                    pbuf[r, pl.ds(j * LANES, LANES)], iot + j * LANES,
                    descending=True)
                ks.append(kj)
                vs.append(vj)
            k01, v01 = _merge16(ks[0], vs[0], ks[1], vs[1])
            k23, v23 = _merge16(ks[2], vs[2], ks[3], vs[3])
            kt, vt = _merge16(k01, v01, k23, v23)
            ssum = jnp.sum(jnp.where(msk, kt, 0.0))
            vn = kt / (ssum + 1e-9)
            rvec = jnp.full((LANES,), r, jnp.int32)
            plsc.store_scatter(vbuf, [rvec, iot], vn, mask=msk)
            plsc.store_scatter(ibuf, [rvec, iot], vt, mask=msk)

        out_base = wid * rows_per_w
        pltpu.sync_copy(ibuf, idx_hbm.at[pl.ds(out_base, rows_per_w), :])
        pltpu.sync_copy(vbuf, val_hbm.at[pl.ds(out_base, rows_per_w), :])

    return sc_topk


@jax.jit
def _router(x_flat, w):
    t, _ = x_flat.shape
    e_dim = w.shape[0]
    sc_topk = {n: _make_sc_topk(n, e_dim) for n in set(CHUNKS)}
    probs_c, idx_c, val_c, zsq_c = [], [], [], []
    tok_base = 0
    for chunk_tokens in CHUNKS:
        p, zs = _softmax_tc_chunk(x_flat, w, tok_base, chunk_tokens)
        tok_base += chunk_tokens
        i_f, v_f = sc_topk[chunk_tokens](p)
        probs_c.append(p)
        idx_c.append(i_f)
        val_c.append(v_f)
        zsq_c.append(zs)
    probs = jnp.concatenate(probs_c, axis=0)
    idx = jnp.concatenate(idx_c, axis=0)
    val = jnp.concatenate(val_c, axis=0)
    zsq = sum(z[0, 0] for z in zsq_c)
    z_loss = Z_LOSS_COEF * zsq / t
    return idx, val, probs, z_loss


def kernel(x_flat, W):
    return _router(x_flat, W)
